# Initial kernel scaffold; baseline (speedup 1.0000x reference)
import jax
import jax.numpy as jnp
from jax.experimental import pallas as pl


def kernel(x, labels, w_stem, b_stem, w_neck, b_neck, bn_gamma, bn_beta, bn_mean, bn_var, w_dec, ada_buffers):
    raise NotImplementedError("write your pallas kernel here")


# speedup vs baseline: 1.6296x; 1.6296x over previous
import jax
import jax.numpy as jnp
from jax import lax
from jax.experimental import pallas as pl
from jax.experimental.pallas import tpu as pltpu

S = 30.0
M = 0.7
H_PARAM = 0.5
ADA_EPS = 1e-6
T_ALPHA = 0.01
GEM_P = 3.0
GEM_EPS = 1e-6
BN_EPS = 1e-5

_VMEM_LIMIT = 56 * 1024 * 1024


def _make_stem_kernel(hw):
    inv_hw = 1.0 / float(hw)

    def _body(x_ref, w_ref, o_ref):
        x = x_ref[0]
        ones = jnp.ones((1, x.shape[1]), jnp.float32)
        x4 = jnp.concatenate([x, ones], axis=0)
        feat = jnp.dot(w_ref[...], x4, preferred_element_type=jnp.float32)
        m = jnp.maximum(feat, GEM_EPS)
        acc = jnp.sum(m * m * m, axis=1, keepdims=True)
        mean = acc * inv_hw
        o_ref[0] = jnp.exp(jnp.log(mean) * (1.0 / GEM_P))

    return _body


def _stem_gem(x_bc_hw, w_stem, b_stem):
    B, C, HW = x_bc_hw.shape
    F = w_stem.shape[0]
    w4 = jnp.concatenate([w_stem, b_stem.reshape(F, 1)], axis=1)

    out = pl.pallas_call(
        _make_stem_kernel(HW),
        grid=(B,),
        in_specs=[
            pl.BlockSpec((1, C, HW), lambda b: (b, 0, 0)),
            pl.BlockSpec((F, C + 1), lambda b: (0, 0)),
        ],
        out_specs=pl.BlockSpec((1, F, 1), lambda b: (b, 0, 0)),
        out_shape=jax.ShapeDtypeStruct((B, F, 1), jnp.float32),
        compiler_params=pltpu.CompilerParams(
            dimension_semantics=("parallel",),
            vmem_limit_bytes=_VMEM_LIMIT,
        ),
        cost_estimate=pl.CostEstimate(
            flops=int(2 * B * HW * (C + 1) * F + 4 * B * HW * F),
            transcendentals=int(2 * B * F),
            bytes_accessed=int(4 * (B * C * HW + F * (C + 1) + B * F)),
        ),
    )(x_bc_hw, w4)
    return out.reshape(B, F)


def _make_decoder_kernel(nj, tnc):
    def _body(pooled_ref, wneck_ref, bneck_ref, gamma_ref, beta_ref,
              rmean_ref, rvar_ref, buf_ref, label_ref, wdec_ref,
              o_ref, stats_ref, z_sc, margin_sc):
        i = pl.program_id(0)
        j = pl.program_id(1)
        nb = pooled_ref.shape[0]

        @pl.when(j == 0)
        def _prologue():
            y = (jnp.dot(pooled_ref[...], wneck_ref[...],
                         preferred_element_type=jnp.float32) + bneck_ref[...])
            y = ((y - rmean_ref[...]) * lax.rsqrt(rvar_ref[...] + BN_EPS)
                 * gamma_ref[...] + beta_ref[...])
            y = 0.5 * y * (1.0 + lax.erf(y * 0.7071067811865476))

            norm = jnp.maximum(jnp.sqrt(jnp.sum(y * y, axis=1, keepdims=True)),
                               ADA_EPS)
            z_sc[...] = y / norm

            bmean = jnp.mean(norm, axis=0, keepdims=True)
            diff = norm - bmean
            denom = float(max(nb - 1, 1))
            bstd = jnp.sqrt(jnp.sum(diff * diff, axis=0, keepdims=True) / denom)
            new_mean = (1.0 - T_ALPHA) * buf_ref[:, 0:1] + T_ALPHA * bmean
            new_std = (1.0 - T_ALPHA) * buf_ref[:, 1:2] + T_ALPHA * bstd
            stats_ref[:, 0:1] = new_mean
            stats_ref[:, 1:2] = new_std
            margin_sc[...] = M + H_PARAM * (norm - new_mean) / (new_std + ADA_EPS)

        w = wdec_ref[...]
        inv_wn = lax.rsqrt(jnp.maximum(jnp.sum(w * w, axis=1, keepdims=True),
                                       1e-24))
        cosine = lax.dot_general(z_sc[...], w * inv_wn,
                                 (((1,), (1,)), ((), ())),
                                 preferred_element_type=jnp.float32)
        cosine = jnp.clip(cosine, -1.0 + ADA_EPS, 1.0 - ADA_EPS)

        cls = (lax.broadcasted_iota(jnp.int32, (nb, tnc), 1)
               + (i * nj + j) * tnc)
        m_ps = margin_sc[...]
        sin_t = jnp.sqrt(jnp.maximum(1.0 - cosine * cosine, 0.0))
        target = cosine * jnp.cos(m_ps) - sin_t * jnp.sin(m_ps)
        o_ref[...] = jnp.where(cls == label_ref[...], target, cosine) * S

    return _body


def _neck_decoder(pooled, w_neck, b_neck, bn_gamma, bn_beta, bn_mean, bn_var,
                  w_dec, ada_buffers, labels):
    B, F = pooled.shape
    E = w_neck.shape[1]
    NC = w_dec.shape[0]
    nc_pad = ((NC + 127) // 128) * 128
    if nc_pad != NC:
        w_dec = jnp.pad(w_dec, ((0, nc_pad - NC), (0, 0)))
    ncores = 2 if nc_pad % 256 == 0 else 1
    tnc = nc_pad // ncores
    for t in (1024, 512, 256, 128):
        if tnc % t == 0:
            tnc = t
            break
    nj = nc_pad // (ncores * tnc)

    args = (
        pooled,
        w_neck,
        b_neck.reshape(1, E),
        bn_gamma.reshape(1, E),
        bn_beta.reshape(1, E),
        bn_mean.reshape(1, E),
        bn_var.reshape(1, E),
        ada_buffers.reshape(1, 2),
        labels.astype(jnp.int32).reshape(B, 1),
        w_dec,
    )
    in_specs = [
        pl.BlockSpec((B, F), lambda i, j: (0, 0)),
        pl.BlockSpec((F, E), lambda i, j: (0, 0)),
        pl.BlockSpec((1, E), lambda i, j: (0, 0)),
        pl.BlockSpec((1, E), lambda i, j: (0, 0)),
        pl.BlockSpec((1, E), lambda i, j: (0, 0)),
        pl.BlockSpec((1, E), lambda i, j: (0, 0)),
        pl.BlockSpec((1, E), lambda i, j: (0, 0)),
        pl.BlockSpec((1, 2), lambda i, j: (0, 0)),
        pl.BlockSpec((B, 1), lambda i, j: (0, 0)),
        pl.BlockSpec((tnc, E), lambda i, j: (i * nj + j, 0)),
    ]
    out_specs = (
        pl.BlockSpec((B, tnc), lambda i, j: (0, i * nj + j)),
        pl.BlockSpec((1, 2), lambda i, j: (0, 0)),
    )
    out_shape = (
        jax.ShapeDtypeStruct((B, nc_pad), jnp.float32),
        jax.ShapeDtypeStruct((1, 2), jnp.float32),
    )
    logits, new_buffers = pl.pallas_call(
        _make_decoder_kernel(nj, tnc),
        grid=(ncores, nj),
        in_specs=in_specs,
        out_specs=out_specs,
        out_shape=out_shape,
        scratch_shapes=[pltpu.VMEM((B, E), jnp.float32),
                        pltpu.VMEM((B, 1), jnp.float32)],
        compiler_params=pltpu.CompilerParams(
            dimension_semantics=("parallel", "arbitrary"),
            vmem_limit_bytes=_VMEM_LIMIT,
        ),
        cost_estimate=pl.CostEstimate(
            flops=int(2 * B * F * E + 3 * nc_pad * E + 2 * B * E * nc_pad
                      + 10 * B * nc_pad),
            transcendentals=int(2 * B * E + nc_pad + 8 * B),
            bytes_accessed=int(4 * (B * F + F * E + 6 * E + B + nc_pad * E
                                    + B * nc_pad + 4)),
        ),
    )(*args)
    return logits[:, :NC], new_buffers


def kernel(x, labels, w_stem, b_stem, w_neck, b_neck, bn_gamma, bn_beta,
           bn_mean, bn_var, w_dec, ada_buffers):
    B, C, Himg, Wimg = x.shape
    x = x.reshape(B, C, Himg * Wimg)
    pooled = _stem_gem(x, w_stem, b_stem)
    if labels is None:
        labels = jnp.full((B,), -1, dtype=jnp.int32)
    return _neck_decoder(pooled, w_neck, b_neck, bn_gamma, bn_beta, bn_mean,
                         bn_var, w_dec, ada_buffers, labels)
